# Initial kernel scaffold; baseline (speedup 1.0000x reference)
import jax
import jax.numpy as jnp
from jax.experimental import pallas as pl


def kernel(z, query_idx, neighbor_idx, edge_times, current_time, t_s, t_e, edge_index, k_core, omega, phi):
    raise NotImplementedError("write your pallas kernel here")


# speedup vs baseline: 1.9747x; 1.9747x over previous
import functools

import jax
import jax.numpy as jnp
from jax import lax
from jax.experimental import pallas as pl
from jax.experimental.pallas import tpu as pltpu
from jax.experimental.pallas import tpu_sc as plsc

_NC = 2
_NS = 16
_L = 16
_NW = _NC * _NS


def _make_sc_fn(n_nodes, e_pad, e_per_w, n_nbr, n_gidx):
    mesh = plsc.VectorSubcoreMesh(core_axis_name="c", subcore_axis_name="s")

    @functools.partial(
        pl.kernel,
        mesh=mesh,
        compiler_params=pltpu.CompilerParams(needs_layout_passes=False),
        out_type=[
            jax.ShapeDtypeStruct((e_pad,), jnp.int32),
            jax.ShapeDtypeStruct((n_gidx,), jnp.int32),
        ],
        scratch_types=[
            pltpu.VMEM((n_nodes,), jnp.int32),
            pltpu.VMEM((n_nbr,), jnp.int32),
            pltpu.VMEM((e_per_w,), jnp.int32),
            pltpu.VMEM((e_per_w,), jnp.int32),
            pltpu.VMEM((e_per_w,), jnp.int32),
            pltpu.VMEM((n_nodes,), jnp.int32),
            pltpu.VMEM((n_gidx,), jnp.int32),
            pltpu.VMEM((n_gidx,), jnp.int32),
        ],
    )
    def sc_fn(nbr_hbm, src_hbm, dst_hbm, kcore_hbm, gidx_hbm, zeros_hbm,
              mask_hbm, cores_hbm,
              table_v, nbr_v, src_v, dst_v, out_v, kc_v, gidx_v, cores_v):
        wid = lax.axis_index("s") * _NC + lax.axis_index("c")
        base = wid * e_per_w

        pltpu.sync_copy(zeros_hbm, table_v)
        pltpu.sync_copy(nbr_hbm, nbr_v)
        ones = jnp.full((_L,), 1, jnp.int32)
        for i in range(n_nbr // _L):
            plsc.store_scatter(table_v, [nbr_v[pl.ds(i * _L, _L)]], ones)

        pltpu.sync_copy(src_hbm.at[pl.ds(base, e_per_w)], src_v)
        pltpu.sync_copy(dst_hbm.at[pl.ds(base, e_per_w)], dst_v)

        def body(i, carry):
            off = i * _L
            ms = plsc.load_gather(table_v, [src_v[pl.ds(off, _L)]])
            md = plsc.load_gather(table_v, [dst_v[pl.ds(off, _L)]])
            out_v[pl.ds(off, _L)] = ms * md
            return carry

        lax.fori_loop(0, e_per_w // _L, body, 0)
        pltpu.sync_copy(out_v, mask_hbm.at[pl.ds(base, e_per_w)])

        @pl.when(wid == 0)
        def _():
            pltpu.sync_copy(kcore_hbm, kc_v)
            pltpu.sync_copy(gidx_hbm, gidx_v)
            for i in range(n_gidx // _L):
                off = i * _L
                cores_v[pl.ds(off, _L)] = plsc.load_gather(
                    kc_v, [gidx_v[pl.ds(off, _L)]])
            pltpu.sync_copy(cores_v, cores_hbm)

    return sc_fn


def _tc_body(t_ref, m_ref, om_ref, ph_ref, ts_ref, te_ref, kc_ref, nc_ref,
             qc_ref, out_ref, accw, accn, cnt):
    i = pl.program_id(0)
    nblk = pl.num_programs(0)

    @pl.when(i == 0)
    def _():
        accw[...] = jnp.zeros_like(accw)
        accn[...] = jnp.zeros_like(accn)
        cnt[0] = 0.0
        cnt[1] = 0.0

    t = t_ref[0]
    ts = ts_ref[0, 0]
    te = te_ref[0, 0]
    tm = ((t >= ts) & (t <= te)).astype(jnp.float32)
    nm = tm * m_ref[0].astype(jnp.float32)

    a = om_ref[...] * t + ph_ref[...]
    rows = lax.broadcasted_iota(jnp.int32, a.shape, 0)
    p = jnp.tanh(jnp.where(rows == 0, a, jnp.sin(a)))

    accw[...] += jnp.sum(p * tm, axis=1, keepdims=True)
    accn[...] += jnp.sum(p * nm, axis=1, keepdims=True)
    cnt[0] += jnp.sum(tm)
    cnt[1] += jnp.sum(nm)

    @pl.when(i == nblk - 1)
    def _():
        nw = cnt[0]
        nn = cnt[1]
        sw = accw[...] / jnp.maximum(nw, 1.0)
        sn = accn[...] / jnp.maximum(nn, 1.0)
        aa = sw / jnp.maximum(jnp.sqrt(jnp.sum(sw * sw)), 1e-12)
        bb = sn / jnp.maximum(jnp.sqrt(jnp.sum(sn * sn)), 1e-12)
        lt = jnp.mean((aa - bb) ** 2)
        lt = jnp.where((nw > 0) & (nn > 0), lt, 0.0)
        kc = kc_ref[...].astype(jnp.float32)
        scale = jnp.max(kc) - jnp.min(kc) + 1e-8
        qv = qc_ref[0, 0].astype(jnp.float32)
        ncf = nc_ref[...].astype(jnp.float32)
        lc = jnp.mean(((ncf - qv) / scale) ** 2)
        out_ref[...] = jnp.broadcast_to(0.5 * lt + 0.5 * lc, (1, 1))


def kernel(z, query_idx, neighbor_idx, edge_times, current_time, t_s, t_e,
           edge_index, k_core, omega, phi):
    del z, current_time
    E = edge_times.shape[0]
    NN = k_core.shape[0]
    NB = neighbor_idx.shape[0]
    F = omega.shape[0]

    e_per_w = ((E + _NW - 1) // _NW + _L - 1) // _L * _L
    e_pad = e_per_w * _NW

    src = edge_index[0].astype(jnp.int32)
    dst = edge_index[1].astype(jnp.int32)
    pad = e_pad - E
    src_p = jnp.pad(src, (0, pad))
    dst_p = jnp.pad(dst, (0, pad))
    n_gidx = NB + _L
    gidx = jnp.concatenate([
        neighbor_idx.astype(jnp.int32),
        jnp.full((_L,), query_idx, dtype=jnp.int32),
    ])
    zeros_tab = jnp.zeros((NN,), jnp.int32)

    sc_fn = _make_sc_fn(NN, e_pad, e_per_w, NB, n_gidx)
    mask_pad, cores = sc_fn(neighbor_idx.astype(jnp.int32), src_p, dst_p,
                            k_core.astype(jnp.int32), gidx, zeros_tab)
    mask = mask_pad[:E]

    B = 1280
    nblk = E // B
    t3 = edge_times.reshape(nblk, 1, B)
    m3 = mask.reshape(nblk, 1, B)
    om2 = omega.reshape(F, 1)
    ph2 = phi.reshape(F, 1)
    tsv = jnp.asarray(t_s, jnp.float32).reshape(1, 1)
    tev = jnp.asarray(t_e, jnp.float32).reshape(1, 1)
    kc2 = k_core.reshape(80, NN // 80)
    nc2 = cores[:NB].reshape(8, NB // 8)
    qc2 = cores[NB:].reshape(1, _L)

    out = pl.pallas_call(
        _tc_body,
        grid=(nblk,),
        in_specs=[
            pl.BlockSpec((1, 1, B), lambda i: (i, 0, 0)),
            pl.BlockSpec((1, 1, B), lambda i: (i, 0, 0)),
            pl.BlockSpec((F, 1), lambda i: (0, 0)),
            pl.BlockSpec((F, 1), lambda i: (0, 0)),
            pl.BlockSpec((1, 1), lambda i: (0, 0)),
            pl.BlockSpec((1, 1), lambda i: (0, 0)),
            pl.BlockSpec(kc2.shape, lambda i: (0, 0)),
            pl.BlockSpec(nc2.shape, lambda i: (0, 0)),
            pl.BlockSpec((1, _L), lambda i: (0, 0)),
        ],
        out_specs=pl.BlockSpec((1, 1), lambda i: (0, 0)),
        out_shape=jax.ShapeDtypeStruct((1, 1), jnp.float32),
        scratch_shapes=[
            pltpu.VMEM((F, 1), jnp.float32),
            pltpu.VMEM((F, 1), jnp.float32),
            pltpu.SMEM((2,), jnp.float32),
        ],
    )(t3, m3, om2, ph2, tsv, tev, kc2, nc2, qc2)
    return out.reshape(())
